# baseline (speedup 1.0000x reference)
import functools

import jax
import jax.numpy as jnp
from jax import lax
from jax.experimental import pallas as pl
from jax.experimental.pallas import tpu as pltpu
from jax.experimental.pallas import tpu_sc as plsc

_DEFAULT_VALUE = 0

_L = 16
_NC = 2
_NS = 16
_NW = _NC * _NS

_N = 16384 * 26
_PER_W = _N // _NW
_VECS = _PER_W // _L
_UNROLL = 16


def _lookup_sc(a_flat, key16, val16):
    mesh = plsc.VectorSubcoreMesh(core_axis_name="c", subcore_axis_name="s")

    @functools.partial(
        pl.kernel,
        mesh=mesh,
        out_type=jax.ShapeDtypeStruct((_N,), jnp.int32),
        scratch_types=[
            pltpu.VMEM((_PER_W,), jnp.int32),
            pltpu.VMEM((_PER_W,), jnp.int32),
            pltpu.VMEM((_L,), jnp.int32),
            pltpu.VMEM((_L,), jnp.int32),
            pltpu.SemaphoreType.DMA,
        ],
    )
    def _k(a_hbm, key_hbm, val_hbm, out_hbm, a_v, o_v, key_v, val_v, sem):
        wid = lax.axis_index("s") * _NC + lax.axis_index("c")
        base = wid * _PER_W
        c_a = pltpu.async_copy(a_hbm.at[pl.ds(base, _PER_W)], a_v, sem)
        c_k = pltpu.async_copy(key_hbm, key_v, sem)
        c_v = pltpu.async_copy(val_hbm, val_v, sem)
        c_a.wait()
        c_k.wait()
        c_v.wait()
        key = key_v[...]
        val = val_v[...]
        default = jnp.full((_L,), _DEFAULT_VALUE, jnp.int32)

        def body(i, carry):
            b = i * (_L * _UNROLL)
            for u in range(_UNROLL):
                x = a_v[pl.ds(b + u * _L, _L)]
                o_v[pl.ds(b + u * _L, _L)] = jnp.where(x == key, val, default)
            return carry

        lax.fori_loop(0, _VECS // _UNROLL, body, 0)
        pltpu.sync_copy(o_v, out_hbm.at[pl.ds(base, _PER_W)])

    return _k(a_flat, key16, val16)


def kernel(a, table_keys, table_values):
    a_flat = jnp.reshape(a, (-1,)).astype(jnp.int32)
    key16 = jnp.broadcast_to(table_keys.astype(jnp.int32), (_L,))
    val16 = jnp.broadcast_to(table_values.astype(jnp.int32), (_L,))
    out = _lookup_sc(a_flat, key16, val16)
    return {"y_click": jnp.reshape(out, a.shape)}


# speedup vs baseline: 1.0072x; 1.0072x over previous
import functools

import jax
import jax.numpy as jnp
from jax import lax
from jax.experimental import pallas as pl
from jax.experimental.pallas import tpu as pltpu
from jax.experimental.pallas import tpu_sc as plsc

_DEFAULT_VALUE = 0

_L = 16
_NC = 1
_NS = 16
_NW = _NC * _NS

_N = 16384 * 26
_PER_W = _N // _NW
_VECS = _PER_W // _L
_UNROLL = 16


def _lookup_sc(a_flat, key16, val16):
    mesh = plsc.VectorSubcoreMesh(core_axis_name="c", subcore_axis_name="s",
                                  num_cores=_NC)

    @functools.partial(
        pl.kernel,
        mesh=mesh,
        out_type=jax.ShapeDtypeStruct((_N,), jnp.int32),
        scratch_types=[
            pltpu.VMEM((_PER_W,), jnp.int32),
            pltpu.VMEM((_PER_W,), jnp.int32),
            pltpu.VMEM((_L,), jnp.int32),
            pltpu.VMEM((_L,), jnp.int32),
            pltpu.SemaphoreType.DMA,
        ],
    )
    def _k(a_hbm, key_hbm, val_hbm, out_hbm, a_v, o_v, key_v, val_v, sem):
        wid = lax.axis_index("s") * _NC + lax.axis_index("c")
        base = wid * _PER_W
        c_a = pltpu.async_copy(a_hbm.at[pl.ds(base, _PER_W)], a_v, sem)
        c_k = pltpu.async_copy(key_hbm, key_v, sem)
        c_v = pltpu.async_copy(val_hbm, val_v, sem)
        c_a.wait()
        c_k.wait()
        c_v.wait()
        key = key_v[...]
        val = val_v[...]
        default = jnp.full((_L,), _DEFAULT_VALUE, jnp.int32)

        def body(i, carry):
            b = i * (_L * _UNROLL)
            for u in range(_UNROLL):
                x = a_v[pl.ds(b + u * _L, _L)]
                o_v[pl.ds(b + u * _L, _L)] = jnp.where(x == key, val, default)
            return carry

        lax.fori_loop(0, _VECS // _UNROLL, body, 0)
        pltpu.sync_copy(o_v, out_hbm.at[pl.ds(base, _PER_W)])

    return _k(a_flat, key16, val16)


def kernel(a, table_keys, table_values):
    a_flat = jnp.reshape(a, (-1,)).astype(jnp.int32)
    key16 = jnp.broadcast_to(table_keys.astype(jnp.int32), (_L,))
    val16 = jnp.broadcast_to(table_values.astype(jnp.int32), (_L,))
    out = _lookup_sc(a_flat, key16, val16)
    return {"y_click": jnp.reshape(out, a.shape)}


# speedup vs baseline: 1.4511x; 1.4408x over previous
import functools

import jax
import jax.numpy as jnp
from jax import lax
from jax.experimental import pallas as pl
from jax.experimental.pallas import tpu as pltpu
from jax.experimental.pallas import tpu_sc as plsc

_DEFAULT_VALUE = 0

_L = 16
_NC = 2
_NS = 16
_NW = _NC * _NS

_R = 16384
_C = 26
_ROWS_W = _R // _NW
_UNROLL = 8


def _lookup_sc(a3, key16, val16):
    mesh = plsc.VectorSubcoreMesh(core_axis_name="c", subcore_axis_name="s")

    @functools.partial(
        pl.kernel,
        mesh=mesh,
        out_type=jax.ShapeDtypeStruct((_NW, _ROWS_W, _C), jnp.int32),
        scratch_types=[
            pltpu.VMEM((_ROWS_W, _C), jnp.int32),
            pltpu.VMEM((_L,), jnp.int32),
            pltpu.VMEM((_L,), jnp.int32),
            pltpu.SemaphoreType.DMA,
        ],
    )
    def _k(a_hbm, key_hbm, val_hbm, out_hbm, a_v, key_v, val_v, sem):
        wid = lax.axis_index("s") * _NC + lax.axis_index("c")
        c_a = pltpu.async_copy(a_hbm.at[wid], a_v, sem)
        c_k = pltpu.async_copy(key_hbm, key_v, sem)
        c_v = pltpu.async_copy(val_hbm, val_v, sem)
        c_a.wait()
        c_k.wait()
        c_v.wait()
        key = key_v[...]
        val = val_v[...]
        default = jnp.full((_L,), _DEFAULT_VALUE, jnp.int32)

        def do(r):
            x0 = a_v[r, pl.ds(0, _L)]
            x1 = a_v[r, pl.ds(_C - _L, _L)]
            a_v[r, pl.ds(0, _L)] = jnp.where(x0 == key, val, default)
            a_v[r, pl.ds(_C - _L, _L)] = jnp.where(x1 == key, val, default)

        def body(i, carry):
            r0 = i * _UNROLL
            for u in range(_UNROLL):
                do(r0 + u)
            return carry

        lax.fori_loop(0, _ROWS_W // _UNROLL, body, 0)
        pltpu.sync_copy(a_v, out_hbm.at[wid])

    return _k(a3, key16, val16)


def kernel(a, table_keys, table_values):
    a3 = jnp.reshape(a, (_NW, _ROWS_W, _C)).astype(jnp.int32)
    key16 = jnp.broadcast_to(table_keys.astype(jnp.int32), (_L,))
    val16 = jnp.broadcast_to(table_values.astype(jnp.int32), (_L,))
    out = _lookup_sc(a3, key16, val16)
    return {"y_click": jnp.reshape(out, (_R, _C))}


# speedup vs baseline: 1.4531x; 1.0014x over previous
import functools

import jax
import jax.numpy as jnp
from jax import lax
from jax.experimental import pallas as pl
from jax.experimental.pallas import tpu as pltpu
from jax.experimental.pallas import tpu_sc as plsc

_DEFAULT_VALUE = 0

_L = 16
_NC = 2
_NS = 16
_NW = _NC * _NS

_R = 16384
_C = 26
_ROWS_W = _R // _NW
_UNROLL = 8


def _lookup_sc(a3, key16, val16):
    mesh = plsc.VectorSubcoreMesh(core_axis_name="c", subcore_axis_name="s")

    @functools.partial(
        pl.kernel,
        mesh=mesh,
        compiler_params=pltpu.CompilerParams(use_tc_tiling_on_sc=True),
        out_type=jax.ShapeDtypeStruct((_NW, _ROWS_W, _C), jnp.int32),
        scratch_types=[
            pltpu.VMEM((_ROWS_W, _C), jnp.int32),
            pltpu.VMEM((_L,), jnp.int32),
            pltpu.VMEM((_L,), jnp.int32),
            pltpu.SemaphoreType.DMA,
        ],
    )
    def _k(a_hbm, key_hbm, val_hbm, out_hbm, a_v, key_v, val_v, sem):
        wid = lax.axis_index("s") * _NC + lax.axis_index("c")
        c_a = pltpu.async_copy(a_hbm.at[wid], a_v, sem)
        c_k = pltpu.async_copy(key_hbm, key_v, sem)
        c_v = pltpu.async_copy(val_hbm, val_v, sem)
        c_a.wait()
        c_k.wait()
        c_v.wait()
        key = key_v[...]
        val = val_v[...]
        default = jnp.full((_L,), _DEFAULT_VALUE, jnp.int32)

        def do(r):
            x0 = a_v[r, pl.ds(0, _L)]
            x1 = a_v[r, pl.ds(_C - _L, _L)]
            a_v[r, pl.ds(0, _L)] = jnp.where(x0 == key, val, default)
            a_v[r, pl.ds(_C - _L, _L)] = jnp.where(x1 == key, val, default)

        def body(i, carry):
            r0 = i * _UNROLL
            for u in range(_UNROLL):
                do(r0 + u)
            return carry

        lax.fori_loop(0, _ROWS_W // _UNROLL, body, 0)
        pltpu.sync_copy(a_v, out_hbm.at[wid])

    return _k(a3, key16, val16)


def kernel(a, table_keys, table_values):
    a3 = jnp.reshape(a, (_NW, _ROWS_W, _C)).astype(jnp.int32)
    key16 = jnp.broadcast_to(table_keys.astype(jnp.int32), (_L,))
    val16 = jnp.broadcast_to(table_values.astype(jnp.int32), (_L,))
    out = _lookup_sc(a3, key16, val16)
    return {"y_click": jnp.reshape(out, (_R, _C))}
